# Initial kernel scaffold; baseline (speedup 1.0000x reference)
import functools

import jax
import jax.numpy as jnp
from jax import lax
from jax.experimental import pallas as pl
from jax.experimental.pallas import tpu as pltpu
from jax.experimental.pallas import tpu_sc as plsc

IDX_MINOR = 128
SUB_PER_CHUNK = 8


def _gather_call(n_rows, n_idx, d):
    info = plsc.get_sparse_core_info()
    nc, ns = info.num_cores, info.num_subcores
    nw = nc * ns
    chunk = SUB_PER_CHUNK * IDX_MINOR
    assert n_idx % (nw * chunk) == 0
    per_w = n_idx // nw
    n_chunks = per_w // chunk

    mesh = plsc.VectorSubcoreMesh(core_axis_name="c", subcore_axis_name="s")

    @functools.partial(
        pl.kernel,
        mesh=mesh,
        out_type=jax.ShapeDtypeStruct((n_idx, d), jnp.float32),
        scratch_types=[
            pltpu.VMEM((SUB_PER_CHUNK, IDX_MINOR), jnp.int32),
            pltpu.VMEM((chunk, d), jnp.float32),
            pltpu.SemaphoreType.DMA,
        ],
    )
    def run(idx_hbm, table_hbm, out_hbm, idx_v, rows_v, sem):
        wid = lax.axis_index("s") * nc + lax.axis_index("c")
        base = wid * per_w

        def body(c, carry):
            off = base + c * chunk
            pltpu.sync_copy(idx_hbm.at[pl.ds(off // IDX_MINOR, SUB_PER_CHUNK)],
                            idx_v)
            copies = []
            for j in range(SUB_PER_CHUNK):
                copies.append(pltpu.async_copy(
                    table_hbm.at[idx_v.at[j]],
                    rows_v.at[pl.ds(j * IDX_MINOR, IDX_MINOR)],
                    sem))
            for c_h in copies:
                c_h.wait()
            pltpu.sync_copy(rows_v, out_hbm.at[pl.ds(off, chunk)])
            return carry

        lax.fori_loop(0, n_chunks, body, 0)

    return run


def kernel(word_ids, embed_table):
    b0, b1 = word_ids.shape
    n_rows, d = embed_table.shape
    n_idx = b0 * b1
    idx2d = word_ids.reshape(n_idx // IDX_MINOR, IDX_MINOR).astype(jnp.int32)
    out = _gather_call(n_rows, n_idx, d)(idx2d, embed_table)
    return out.reshape(b0, b1, d)


# speedup vs baseline: 2.6677x; 2.6677x over previous
import functools

import jax
import jax.numpy as jnp
from jax import lax
from jax.experimental import pallas as pl
from jax.experimental.pallas import tpu as pltpu
from jax.experimental.pallas import tpu_sc as plsc

IDX_MINOR = 128
SUB_PER_CHUNK = 8
D_PAD = 112


def _gather_call(n_idx):
    info = plsc.get_sparse_core_info()
    nc, ns = info.num_cores, info.num_subcores
    nw = nc * ns
    chunk = SUB_PER_CHUNK * IDX_MINOR
    assert n_idx % (nw * chunk) == 0
    per_w = n_idx // nw
    n_chunks = per_w // chunk

    mesh = plsc.VectorSubcoreMesh(core_axis_name="c", subcore_axis_name="s")

    @functools.partial(
        pl.kernel,
        mesh=mesh,
        out_type=jax.ShapeDtypeStruct((n_idx, D_PAD), jnp.float32),
        compiler_params=pltpu.CompilerParams(use_tc_tiling_on_sc=False),
        scratch_types=[
            pltpu.VMEM((SUB_PER_CHUNK, IDX_MINOR), jnp.int32),
            pltpu.VMEM((chunk, D_PAD), jnp.float32),
            pltpu.SemaphoreType.DMA,
        ],
    )
    def run(idx_hbm, table_hbm, out_hbm, idx_v, rows_v, sem):
        wid = lax.axis_index("s") * nc + lax.axis_index("c")
        base = wid * per_w

        def body(c, carry):
            off = pl.multiple_of(base + c * chunk, chunk)
            row_off = pl.multiple_of(off // IDX_MINOR, SUB_PER_CHUNK)
            pltpu.sync_copy(idx_hbm.at[pl.ds(row_off, SUB_PER_CHUNK)],
                            idx_v)
            copies = []
            for j in range(SUB_PER_CHUNK):
                copies.append(pltpu.async_copy(
                    table_hbm.at[idx_v.at[j]],
                    rows_v.at[pl.ds(j * IDX_MINOR, IDX_MINOR)],
                    sem))
            for c_h in copies:
                c_h.wait()
            pltpu.sync_copy(rows_v, out_hbm.at[pl.ds(off, chunk)])
            return carry

        lax.fori_loop(0, n_chunks, body, 0)

    return run


def kernel(word_ids, embed_table):
    b0, b1 = word_ids.shape
    n_rows, d = embed_table.shape
    n_idx = b0 * b1
    idx2d = word_ids.reshape(n_idx // IDX_MINOR, IDX_MINOR).astype(jnp.int32)
    table_p = jnp.pad(embed_table, ((0, 0), (0, D_PAD - d)))
    out_pad = _gather_call(n_idx)(idx2d, table_p)
    return out_pad[:, :d].reshape(b0, b1, d)


# speedup vs baseline: 4.0377x; 1.5135x over previous
import functools

import jax
import jax.numpy as jnp
from jax import lax
from jax.experimental import pallas as pl
from jax.experimental.pallas import tpu as pltpu
from jax.experimental.pallas import tpu_sc as plsc

IDX_MINOR = 128
SUB_PER_CHUNK = 4
D_PAD = 128


def _gather_call(n_idx):
    info = plsc.get_sparse_core_info()
    nc, ns = info.num_cores, info.num_subcores
    nw = nc * ns
    chunk = SUB_PER_CHUNK * IDX_MINOR
    assert n_idx % (nw * chunk) == 0
    per_w = n_idx // nw
    n_chunks = per_w // chunk

    mesh = plsc.VectorSubcoreMesh(core_axis_name="c", subcore_axis_name="s")

    @functools.partial(
        pl.kernel,
        mesh=mesh,
        out_type=jax.ShapeDtypeStruct((n_idx, D_PAD), jnp.float32),
        scratch_types=[
            pltpu.VMEM((chunk,), jnp.int32),
            pltpu.VMEM((chunk, D_PAD), jnp.float32),
            pltpu.SemaphoreType.DMA,
        ],
    )
    def run(idx_hbm, table_hbm, out_hbm, idx_v, rows_v, sem):
        wid = lax.axis_index("s") * nc + lax.axis_index("c")
        base = wid * per_w

        def body(c, carry):
            off = pl.multiple_of(base + c * chunk, chunk)
            pltpu.sync_copy(idx_hbm.at[pl.ds(off, chunk)], idx_v)
            copies = []
            for j in range(SUB_PER_CHUNK):
                copies.append(pltpu.async_copy(
                    table_hbm.at[idx_v.at[pl.ds(j * IDX_MINOR, IDX_MINOR)]],
                    rows_v.at[pl.ds(j * IDX_MINOR, IDX_MINOR)],
                    sem))
            for c_h in copies:
                c_h.wait()
            pltpu.sync_copy(rows_v, out_hbm.at[pl.ds(off, chunk)])
            return carry

        lax.fori_loop(0, n_chunks, body, 0)

    return run


def kernel(word_ids, embed_table):
    b0, b1 = word_ids.shape
    n_rows, d = embed_table.shape
    n_idx = b0 * b1
    idx1d = word_ids.reshape(-1).astype(jnp.int32)
    table_p = jnp.pad(embed_table, ((0, 0), (0, D_PAD - d)))
    out_pad = _gather_call(n_idx)(idx1d, table_p)
    return out_pad[:, :d].reshape(b0, b1, d)
